# baseline (speedup 1.0000x reference)
import functools

import jax
import jax.numpy as jnp
from jax import lax
from jax.experimental import pallas as pl
from jax.experimental.pallas import tpu as pltpu
from jax.experimental.pallas import tpu_sc as plsc

EMBED_DIM = 64
VOCAB = 102
NC, NS = 2, 16
NW = NC * NS
CHUNK = 800
LANES = 16


def _make_lookup(total_rows: int):
    per_w = total_rows // NW
    nchunk = per_w // CHUNK
    npair = nchunk // 2
    groups = CHUNK // LANES
    mesh = plsc.VectorSubcoreMesh(core_axis_name="c", subcore_axis_name="s")

    @functools.partial(
        pl.kernel,
        mesh=mesh,
        out_type=jax.ShapeDtypeStruct((total_rows * EMBED_DIM,), jnp.float32),
        compiler_params=pltpu.CompilerParams(
            use_tc_tiling_on_sc=False, needs_layout_passes=False
        ),
        scratch_types=[
            pltpu.VMEM((VOCAB * EMBED_DIM,), jnp.float32),
            pltpu.VMEM((2, CHUNK), jnp.int32),
            pltpu.VMEM((2, CHUNK * EMBED_DIM), jnp.float32),
            pltpu.SemaphoreType.DMA,
            pltpu.SemaphoreType.DMA,
            pltpu.SemaphoreType.DMA,
            pltpu.SemaphoreType.DMA,
        ],
    )
    def lookup(x_hbm, tab_hbm, out_hbm, tab_v, idx_v, rows_v, i0, i1, o0, o1):
        wid = lax.axis_index("s") * NC + lax.axis_index("c")
        row_base = wid * per_w
        ibuf = [idx_v.at[0], idx_v.at[1]]
        rbuf = [rows_v.at[0], rows_v.at[1]]
        isem = [i0, i1]
        osem = [o0, o1]

        def idx_desc(c, buf):
            off = pl.multiple_of(row_base + c * CHUNK, 8)
            return pltpu.make_async_copy(
                x_hbm.at[pl.ds(off, CHUNK)], ibuf[buf], isem[buf]
            )

        def out_desc(c, buf):
            off = pl.multiple_of((row_base + c * CHUNK) * EMBED_DIM, 8)
            return pltpu.make_async_copy(
                rbuf[buf], out_hbm.at[pl.ds(off, CHUNK * EMBED_DIM)], osem[buf]
            )

        def compute(buf):
            idx_ref = ibuf[buf]
            rows_ref = rbuf[buf]
            lanes64 = lax.iota(jnp.int32, LANES) * EMBED_DIM

            def group(g, _):
                x16 = idx_ref[pl.ds(g * LANES, LANES)]
                src = x16 * EMBED_DIM
                dst = g * (LANES * EMBED_DIM) + lanes64
                for col in range(EMBED_DIM):
                    v = plsc.load_gather(tab_v, [src + col])
                    plsc.store_scatter(rows_ref, [dst + col], v)
                return ()

            lax.fori_loop(0, groups, group, (), unroll=False)

        pltpu.sync_copy(tab_hbm, tab_v)
        idx_desc(0, 0).start()
        idx_desc(1, 1).start()

        def pair(p, _):
            for side in range(2):
                c = 2 * p + side
                idx_desc(c, side).wait()

                @pl.when(p > 0)
                def _():
                    out_desc(c - 2, side).wait()

                compute(side)

                @pl.when(p < npair - 1)
                def _():
                    idx_desc(c + 2, side).start()

                out_desc(c, side).start()
            return ()

        lax.fori_loop(0, npair, pair, (), unroll=False)
        out_desc(nchunk - 2, 0).wait()
        out_desc(nchunk - 1, 1).wait()

    return lookup


def kernel(x, token_embedding):
    B, L = x.shape
    total = B * L
    idx = x.astype(jnp.int32).reshape(total)
    tab = token_embedding.reshape(VOCAB * EMBED_DIM)
    out = _make_lookup(total)(idx, tab)
    return out.reshape(B, L, EMBED_DIM)


# speedup vs baseline: 3.7637x; 3.7637x over previous
import functools

import jax
import jax.numpy as jnp
from jax import lax
from jax.experimental import pallas as pl
from jax.experimental.pallas import tpu as pltpu
from jax.experimental.pallas import tpu_sc as plsc

EMBED_DIM = 64
VOCAB = 102
NC, NS = 2, 16
NW = NC * NS
CHUNK = 800
LANES = 16
UNROLL = 16


def _make_lookup(total_rows: int):
    per_w = total_rows // NW
    nchunk = per_w // CHUNK
    npair = nchunk // 2
    groups = CHUNK // UNROLL
    mesh = plsc.VectorSubcoreMesh(core_axis_name="c", subcore_axis_name="s")

    @functools.partial(
        pl.kernel,
        mesh=mesh,
        out_type=jax.ShapeDtypeStruct((total_rows * EMBED_DIM,), jnp.float32),
        compiler_params=pltpu.CompilerParams(
            use_tc_tiling_on_sc=False, needs_layout_passes=False
        ),
        scratch_types=[
            pltpu.VMEM((VOCAB * EMBED_DIM,), jnp.float32),
            pltpu.VMEM((2, CHUNK), jnp.int32),
            pltpu.VMEM((2, CHUNK * EMBED_DIM), jnp.float32),
            pltpu.SemaphoreType.DMA,
            pltpu.SemaphoreType.DMA,
            pltpu.SemaphoreType.DMA,
            pltpu.SemaphoreType.DMA,
        ],
    )
    def lookup(x_hbm, tab_hbm, out_hbm, tab_v, idx_v, rows_v, i0, i1, o0, o1):
        wid = lax.axis_index("s") * NC + lax.axis_index("c")
        row_base = wid * per_w
        ibuf = [idx_v.at[0], idx_v.at[1]]
        rbuf = [rows_v.at[0], rows_v.at[1]]
        isem = [i0, i1]
        osem = [o0, o1]

        def idx_desc(c, buf):
            off = pl.multiple_of(row_base + c * CHUNK, 8)
            return pltpu.make_async_copy(
                x_hbm.at[pl.ds(off, CHUNK)], ibuf[buf], isem[buf]
            )

        def out_desc(c, buf):
            off = pl.multiple_of((row_base + c * CHUNK) * EMBED_DIM, 8)
            return pltpu.make_async_copy(
                rbuf[buf], out_hbm.at[pl.ds(off, CHUNK * EMBED_DIM)], osem[buf]
            )

        def compute(buf):
            idx_ref = ibuf[buf]
            rows_ref = rbuf[buf]

            def group(g, _):
                r0 = g * UNROLL
                xg = idx_ref[pl.ds(r0, LANES)] * EMBED_DIM
                for u in range(0, UNROLL, 2):
                    src_a, src_b = xg[u], xg[u + 1]
                    dst = (r0 + u) * EMBED_DIM
                    vals = [
                        tab_v[pl.ds(src_a + k, LANES)]
                        for k in range(0, EMBED_DIM, LANES)
                    ] + [
                        tab_v[pl.ds(src_b + k, LANES)]
                        for k in range(0, EMBED_DIM, LANES)
                    ]
                    for i, v in enumerate(vals):
                        rows_ref[pl.ds(dst + i * LANES, LANES)] = v
                return ()

            lax.fori_loop(0, groups, group, (), unroll=False)

        pltpu.sync_copy(tab_hbm, tab_v)
        idx_desc(0, 0).start()
        idx_desc(1, 1).start()

        def pair(p, _):
            for side in range(2):
                c = 2 * p + side
                idx_desc(c, side).wait()

                @pl.when(p > 0)
                def _():
                    out_desc(c - 2, side).wait()

                compute(side)

                @pl.when(p < npair - 1)
                def _():
                    idx_desc(c + 2, side).start()

                out_desc(c, side).start()
            return ()

        lax.fori_loop(0, npair, pair, (), unroll=False)
        out_desc(nchunk - 2, 0).wait()
        out_desc(nchunk - 1, 1).wait()

    return lookup


def kernel(x, token_embedding):
    B, L = x.shape
    total = B * L
    idx = x.astype(jnp.int32).reshape(total)
    tab = token_embedding.reshape(VOCAB * EMBED_DIM)
    out = _make_lookup(total)(idx, tab)
    return out.reshape(B, L, EMBED_DIM)


# speedup vs baseline: 7.3028x; 1.9403x over previous
import functools

import jax
import jax.numpy as jnp
from jax import lax
from jax.experimental import pallas as pl
from jax.experimental.pallas import tpu as pltpu
from jax.experimental.pallas import tpu_sc as plsc

EMBED_DIM = 64
VOCAB = 102
NC, NS = 2, 16
NW = NC * NS
CHUNK = 400
LANES = 16
UNROLL = 16


def _make_lookup(total_rows: int):
    per_w = total_rows // NW
    nchunk = per_w // CHUNK
    npair = nchunk // 2
    groups = CHUNK // UNROLL
    mesh = plsc.VectorSubcoreMesh(core_axis_name="c", subcore_axis_name="s")

    @functools.partial(
        pl.kernel,
        mesh=mesh,
        out_type=jax.ShapeDtypeStruct((total_rows, EMBED_DIM), jnp.float32),
        scratch_types=[
            pltpu.VMEM((VOCAB * EMBED_DIM,), jnp.float32),
            pltpu.VMEM((2 * CHUNK,), jnp.int32),
            pltpu.VMEM((2, CHUNK, EMBED_DIM), jnp.float32),
            pltpu.SemaphoreType.DMA,
            pltpu.SemaphoreType.DMA,
            pltpu.SemaphoreType.DMA,
            pltpu.SemaphoreType.DMA,
        ],
    )
    def lookup(x_hbm, tab_hbm, out_hbm, tab_v, idx_v, rows_v, i0, i1, o0, o1):
        wid = lax.axis_index("s") * NC + lax.axis_index("c")
        row_base = wid * per_w
        ibuf = [idx_v.at[pl.ds(0, CHUNK)], idx_v.at[pl.ds(CHUNK, CHUNK)]]
        rbuf = [rows_v.at[0], rows_v.at[1]]
        isem = [i0, i1]
        osem = [o0, o1]

        def idx_desc(c, buf):
            off = pl.multiple_of(row_base + c * CHUNK, 8)
            return pltpu.make_async_copy(
                x_hbm.at[pl.ds(off, CHUNK)], ibuf[buf], isem[buf]
            )

        def out_desc(c, buf):
            off = pl.multiple_of(row_base + c * CHUNK, 8)
            return pltpu.make_async_copy(
                rbuf[buf], out_hbm.at[pl.ds(off, CHUNK)], osem[buf]
            )

        def compute(buf):
            idx_ref = ibuf[buf]
            rows_ref = rbuf[buf]

            def group(g, _):
                r0 = g * UNROLL
                xg = idx_ref[pl.ds(r0, LANES)] * EMBED_DIM
                for u in range(0, UNROLL, 2):
                    src_a, src_b = xg[u], xg[u + 1]
                    vals = [
                        tab_v[pl.ds(src_a + k, LANES)]
                        for k in range(0, EMBED_DIM, LANES)
                    ] + [
                        tab_v[pl.ds(src_b + k, LANES)]
                        for k in range(0, EMBED_DIM, LANES)
                    ]
                    for i, v in enumerate(vals):
                        r = r0 + u + i // 4
                        rows_ref[r, pl.ds((i % 4) * LANES, LANES)] = v
                return ()

            lax.fori_loop(0, groups, group, (), unroll=False)

        pltpu.sync_copy(tab_hbm, tab_v)
        idx_desc(0, 0).start()
        idx_desc(1, 1).start()

        def pair(p, _):
            for side in range(2):
                c = 2 * p + side
                idx_desc(c, side).wait()

                @pl.when(p > 0)
                def _():
                    out_desc(c - 2, side).wait()

                compute(side)

                @pl.when(p < npair - 1)
                def _():
                    idx_desc(c + 2, side).start()

                out_desc(c, side).start()
            return ()

        lax.fori_loop(0, npair, pair, (), unroll=False)
        out_desc(nchunk - 2, 0).wait()
        out_desc(nchunk - 1, 1).wait()

    return lookup


def kernel(x, token_embedding):
    B, L = x.shape
    total = B * L
    idx = x.astype(jnp.int32).reshape(total)
    tab = token_embedding.reshape(VOCAB * EMBED_DIM)
    out = _make_lookup(total)(idx, tab)
    return out.reshape(B, L, EMBED_DIM)
